# baseline (speedup 1.0000x reference)
import jax
import jax.numpy as jnp
from jax.experimental import pallas as pl
from jax.experimental.pallas import tpu as pltpu

LN_EPS = 1e-5
NEG_BIG = -1e30

_HEADS = 8
_DH = 32
_DEPTH = 4


_ERF_C = (2.3429644791391036e-13, -2.4802636884169648e-11, 1.1863734997912792e-09,
          -3.406008989736303e-08, 6.588825071803538e-07, -9.165608612604688e-06,
          9.587448277226247e-05, -0.0007806333136118556, 0.00507629620340517,
          -0.026687536360499153, 0.11272402506968957, -0.3760975767322681,
          1.128377951450486)


def _gelu(x):
    u = x * 0.7071067811865475
    w = u * u
    p = jnp.full_like(w, _ERF_C[0])
    for c in _ERF_C[1:]:
        p = p * w + c
    e = jnp.clip(u * p, -1.0, 1.0)
    return 0.5 * x * (1.0 + e)


def _layernorm(x, g, b):
    mu = jnp.mean(x, axis=-1, keepdims=True)
    xc = x - mu
    var = jnp.mean(xc * xc, axis=-1, keepdims=True)
    return xc * jax.lax.rsqrt(var + LN_EPS) * g + b


def _corrector_kernel(label_ref, skt_ref, pos_ref, wlab_ref, blab_ref,
                      ln1g_ref, ln1b_ref, wqkv_ref, wo_ref, bo_ref,
                      ln2g_ref, ln2b_ref, w1_ref, b1_ref, w2_ref, b2_ref,
                      gw01_ref, gm_ref, am0_ref, aoffr_ref, gb_ref,
                      o_ref):
    bt = label_ref.shape[0]
    lab_dim = label_ref.shape[2]
    jt = skt_ref.shape[1]
    dm = skt_ref.shape[2]
    np_tok = pos_ref.shape[0]
    n_real = jt + 1
    fdim = gb_ref.shape[-1]
    hnp = _HEADS * np_tok

    lab = label_ref[...].reshape(bt, lab_dim)
    emb = jnp.dot(lab, wlab_ref[...], preferred_element_type=jnp.float32) + blab_ref[...]
    x3 = jnp.concatenate(
        [skt_ref[...], emb[:, None, :], jnp.zeros((bt, np_tok - n_real, dm), jnp.float32)],
        axis=1)
    x = (x3 + pos_ref[...][None]).reshape(bt * np_tok, dm)

    row_i = jax.lax.broadcasted_iota(jnp.int32, (hnp, 2 * dm), 0)
    col_i = jax.lax.broadcasted_iota(jnp.int32, (hnp, 2 * dm), 1)
    kvmask2 = ((row_i // np_tok) == ((col_i % dm) // _DH)).astype(jnp.bfloat16)
    kvmask = kvmask2[:, :dm]
    cb = jax.lax.broadcasted_iota(jnp.int32, (1, 1, hnp), 2)
    colbias = jnp.where((cb % np_tok) < n_real, 0.0, NEG_BIG).astype(jnp.float32)

    for l in range(_DEPTH):
        xn = _layernorm(x, ln1g_ref[l], ln1b_ref[l]).astype(jnp.bfloat16)
        qkv = jnp.dot(xn, wqkv_ref[l], preferred_element_type=jnp.float32)
        qkv3 = qkv.astype(jnp.bfloat16).reshape(bt, np_tok, 3 * dm)
        q = qkv3[..., :dm]
        kv = qkv3[..., dm:]
        kvhat = jnp.concatenate([kv] * _HEADS, axis=1) * kvmask2[None]
        khat = kvhat[..., :dm]
        vhat = kvhat[..., dm:]
        s = jax.lax.dot_general(q, khat, (((2,), (2,)), ((0,), (0,))),
                                preferred_element_type=jnp.float32)
        s = s + colbias
        s = s - jnp.max(s, axis=2, keepdims=True)
        p = jnp.exp(s).astype(jnp.bfloat16)
        pv = jax.lax.dot_general(p, vhat, (((2,), (1,)), ((0,), (0,))),
                                 preferred_element_type=jnp.float32)
        den = jnp.dot(p.reshape(bt * np_tok, hnp), kvmask,
                      preferred_element_type=jnp.float32)
        attn = (pv.reshape(bt * np_tok, dm) / den).astype(jnp.bfloat16)
        x = x + jnp.dot(attn, wo_ref[l], preferred_element_type=jnp.float32) + bo_ref[l]

        xn2 = _layernorm(x, ln2g_ref[l], ln2b_ref[l]).astype(jnp.bfloat16)
        hid = _gelu(jnp.dot(xn2, w1_ref[l], preferred_element_type=jnp.float32) + b1_ref[l])
        x = x + jnp.dot(hid.astype(jnp.bfloat16), w2_ref[l],
                        preferred_element_type=jnp.float32) + b2_ref[l]

    xj = x.reshape(bt, np_tok, dm)[:, :jt, :].reshape(bt * jt, dm)
    h01 = jnp.dot(xj, gw01_ref[...], preferred_element_type=jnp.float32)
    h3 = h01.reshape(bt, jt, 2 * fdim)
    h0 = h3[..., :fdim]
    h1 = h3[..., fdim:]
    mh1 = gm_ref[...][None] * h1
    z = jnp.zeros((bt, jt, fdim), jnp.float32)
    for kk in range(jt):
        z = z + mh1[:, kk:kk + 1, :] * aoffr_ref[kk][None]
    o_ref[...] = am0_ref[...][None] * h0 + z + gb_ref[...][None]


def _full(arr):
    nd = arr.ndim
    return pl.BlockSpec(arr.shape, lambda t, _nd=nd: (0,) * _nd)


def kernel(label, skt, label_emb_w, label_emb_b, pos_embedding, gcn_w0, gcn_w1,
           gcn_m, adj, adj2, gcn_bias, *layer_args):
    layers = [layer_args[i * 11:(i + 1) * 11] for i in range(_DEPTH)]
    (ln1_g, ln1_b, w_qkv, w_o, b_o, ln2_g, ln2_b, w_1, b_1, w_2, b_2) = (
        [jnp.stack([lyr[i] for lyr in layers], axis=0) for i in range(11)])

    B, _, L = label.shape
    J, D = skt.shape[1], skt.shape[2]
    inner = _HEADS * _DH
    N = J + 1
    Np = -(-N // 8) * 8
    F = gcn_bias.shape[-1]

    bt = min(64, B)
    n_tiles = -(-B // bt)
    B_pad = n_tiles * bt

    label = label.astype(jnp.float32)
    skt = skt.astype(jnp.float32)
    if B_pad != B:
        label = jnp.pad(label, ((0, B_pad - B), (0, 0), (0, 0)))
        skt = jnp.pad(skt, ((0, B_pad - B), (0, 0), (0, 0)))

    pos = pos_embedding[0]
    pos_perm = jnp.concatenate([pos[1:], pos[:1]], axis=0)
    pos_pad = jnp.zeros((Np, D), jnp.float32).at[:N].set(pos_perm)
    w_qkv = w_qkv.at[:, :, :inner].multiply(_DH ** -0.5).astype(jnp.bfloat16)
    w_o = w_o.astype(jnp.bfloat16)
    w_1 = w_1.astype(jnp.bfloat16)
    w_2 = w_2.astype(jnp.bfloat16)

    a = adj + adj2
    a_sym = 0.5 * (a + a.T)
    eye = jnp.eye(J, dtype=jnp.float32)
    a_off = a_sym * (1.0 - eye)
    am0 = jnp.diagonal(a_sym)[:, None] * gcn_m
    gw01 = jnp.concatenate([gcn_w0, gcn_w1], axis=1)
    aoff_rep = jnp.broadcast_to(a_off.T[:, :, None], (J, J, F))

    weights = [pos_pad, label_emb_w, label_emb_b,
               ln1_g, ln1_b, w_qkv, w_o, b_o, ln2_g, ln2_b, w_1, b_1, w_2, b_2,
               gw01, gcn_m, am0, aoff_rep, gcn_bias]

    out = pl.pallas_call(
        _corrector_kernel,
        out_shape=jax.ShapeDtypeStruct((B_pad, J, F), jnp.float32),
        grid=(n_tiles,),
        in_specs=[pl.BlockSpec((bt, 1, L), lambda t: (t, 0, 0)),
                  pl.BlockSpec((bt, J, D), lambda t: (t, 0, 0))]
                 + [_full(w) for w in weights],
        out_specs=pl.BlockSpec((bt, J, F), lambda t: (t, 0, 0)),
        compiler_params=pltpu.CompilerParams(dimension_semantics=("parallel",)),
    )(label, skt, *weights)
    if B_pad != B:
        out = out[:B]
    return out


# speedup vs baseline: 1.0925x; 1.0925x over previous
import jax
import jax.numpy as jnp
from jax.experimental import pallas as pl
from jax.experimental.pallas import tpu as pltpu

LN_EPS = 1e-5
NEG_BIG = -1e30

_HEADS = 8
_DH = 32
_DEPTH = 4


def _erf(x):
    a1, a2, a3, a4, a5 = 0.254829592, -0.284496736, 1.421413741, -1.453152027, 1.061405429
    p = 0.3275911
    s = jnp.sign(x)
    ax = jnp.abs(x)
    t = 1.0 / (1.0 + p * ax)
    poly = ((((a5 * t + a4) * t + a3) * t + a2) * t + a1) * t
    return s * (1.0 - poly * jnp.exp(-ax * ax))


def _gelu(x):
    return 0.5 * x * (1.0 + _erf(x * 0.7071067811865475))


def _layernorm(x, g, b):
    mu = jnp.mean(x, axis=-1, keepdims=True)
    xc = x - mu
    var = jnp.mean(xc * xc, axis=-1, keepdims=True)
    return xc * jax.lax.rsqrt(var + LN_EPS) * g + b


def _corrector_kernel(label_ref, skt_ref, pos_ref, wlab_ref, blab_ref,
                      ln1g_ref, ln1b_ref, wqkv_ref, wo_ref, bo_ref,
                      ln2g_ref, ln2b_ref, w1_ref, b1_ref, w2_ref, b2_ref,
                      gw01_ref, gm_ref, am0_ref, aoffr_ref, gb_ref,
                      o_ref):
    bt = label_ref.shape[0]
    lab_dim = label_ref.shape[2]
    jt = skt_ref.shape[1]
    dm = skt_ref.shape[2]
    np_tok = pos_ref.shape[0]
    n_real = jt + 1
    fdim = gb_ref.shape[-1]
    hnp = _HEADS * np_tok

    lab = label_ref[...].reshape(bt, lab_dim)
    emb = jnp.dot(lab, wlab_ref[...], preferred_element_type=jnp.float32) + blab_ref[...]
    x3 = jnp.concatenate(
        [skt_ref[...], emb[:, None, :], jnp.zeros((bt, np_tok - n_real, dm), jnp.float32)],
        axis=1)
    x = (x3 + pos_ref[...][None]).reshape(bt * np_tok, dm)

    row_i = jax.lax.broadcasted_iota(jnp.int32, (hnp, 2 * dm), 0)
    col_i = jax.lax.broadcasted_iota(jnp.int32, (hnp, 2 * dm), 1)
    kvmask2 = ((row_i // np_tok) == ((col_i % dm) // _DH)).astype(jnp.bfloat16)
    kvmask = kvmask2[:, :dm]
    cb = jax.lax.broadcasted_iota(jnp.int32, (1, 1, hnp), 2)
    colbias = jnp.where((cb % np_tok) < n_real, 0.0, NEG_BIG).astype(jnp.float32)

    for l in range(_DEPTH):
        xn = _layernorm(x, ln1g_ref[l], ln1b_ref[l]).astype(jnp.bfloat16)
        qkv = jnp.dot(xn, wqkv_ref[l], preferred_element_type=jnp.float32)
        qkv3 = qkv.astype(jnp.bfloat16).reshape(bt, np_tok, 3 * dm)
        q = qkv3[..., :dm]
        kv = qkv3[..., dm:]
        kvhat = jnp.concatenate([kv] * _HEADS, axis=1) * kvmask2[None]
        khat = kvhat[..., :dm]
        vhat = kvhat[..., dm:]
        s = jax.lax.dot_general(q, khat, (((2,), (2,)), ((0,), (0,))),
                                preferred_element_type=jnp.float32)
        s = s + colbias
        s = s - jnp.max(s, axis=2, keepdims=True)
        p = jnp.exp(s).astype(jnp.bfloat16)
        pv = jax.lax.dot_general(p, vhat, (((2,), (1,)), ((0,), (0,))),
                                 preferred_element_type=jnp.float32)
        den = jnp.dot(p.reshape(bt * np_tok, hnp), kvmask,
                      preferred_element_type=jnp.float32)
        attn = (pv.reshape(bt * np_tok, dm) / den).astype(jnp.bfloat16)
        x = x + jnp.dot(attn, wo_ref[l], preferred_element_type=jnp.float32) + bo_ref[l]

        xn2 = _layernorm(x, ln2g_ref[l], ln2b_ref[l]).astype(jnp.bfloat16)
        hid = _gelu(jnp.dot(xn2, w1_ref[l], preferred_element_type=jnp.float32) + b1_ref[l])
        x = x + jnp.dot(hid.astype(jnp.bfloat16), w2_ref[l],
                        preferred_element_type=jnp.float32) + b2_ref[l]

    xj = x.reshape(bt, np_tok, dm)[:, :jt, :].reshape(bt * jt, dm)
    h01 = jnp.dot(xj, gw01_ref[...], preferred_element_type=jnp.float32)
    h3 = h01.reshape(bt, jt, 2 * fdim)
    h0 = h3[..., :fdim]
    h1 = h3[..., fdim:]
    mh1 = gm_ref[...][None] * h1
    z = jnp.zeros((bt, jt, fdim), jnp.float32)
    for kk in range(jt):
        z = z + mh1[:, kk:kk + 1, :] * aoffr_ref[kk][None]
    o_ref[...] = am0_ref[...][None] * h0 + z + gb_ref[...][None]


def _full(arr):
    nd = arr.ndim
    return pl.BlockSpec(arr.shape, lambda t, _nd=nd: (0,) * _nd)


def kernel(label, skt, label_emb_w, label_emb_b, pos_embedding, gcn_w0, gcn_w1,
           gcn_m, adj, adj2, gcn_bias, *layer_args):
    layers = [layer_args[i * 11:(i + 1) * 11] for i in range(_DEPTH)]
    (ln1_g, ln1_b, w_qkv, w_o, b_o, ln2_g, ln2_b, w_1, b_1, w_2, b_2) = (
        [jnp.stack([lyr[i] for lyr in layers], axis=0) for i in range(11)])

    B, _, L = label.shape
    J, D = skt.shape[1], skt.shape[2]
    inner = _HEADS * _DH
    N = J + 1
    Np = -(-N // 8) * 8
    F = gcn_bias.shape[-1]

    bt = min(64, B)
    n_tiles = -(-B // bt)
    B_pad = n_tiles * bt

    label = label.astype(jnp.float32)
    skt = skt.astype(jnp.float32)
    if B_pad != B:
        label = jnp.pad(label, ((0, B_pad - B), (0, 0), (0, 0)))
        skt = jnp.pad(skt, ((0, B_pad - B), (0, 0), (0, 0)))

    pos = pos_embedding[0]
    pos_perm = jnp.concatenate([pos[1:], pos[:1]], axis=0)
    pos_pad = jnp.zeros((Np, D), jnp.float32).at[:N].set(pos_perm)
    w_qkv = w_qkv.at[:, :, :inner].multiply(_DH ** -0.5).astype(jnp.bfloat16)
    w_o = w_o.astype(jnp.bfloat16)
    w_1 = w_1.astype(jnp.bfloat16)
    w_2 = w_2.astype(jnp.bfloat16)

    a = adj + adj2
    a_sym = 0.5 * (a + a.T)
    eye = jnp.eye(J, dtype=jnp.float32)
    a_off = a_sym * (1.0 - eye)
    am0 = jnp.diagonal(a_sym)[:, None] * gcn_m
    gw01 = jnp.concatenate([gcn_w0, gcn_w1], axis=1)
    aoff_rep = jnp.broadcast_to(a_off.T[:, :, None], (J, J, F))

    weights = [pos_pad, label_emb_w, label_emb_b,
               ln1_g, ln1_b, w_qkv, w_o, b_o, ln2_g, ln2_b, w_1, b_1, w_2, b_2,
               gw01, gcn_m, am0, aoff_rep, gcn_bias]

    out = pl.pallas_call(
        _corrector_kernel,
        out_shape=jax.ShapeDtypeStruct((B_pad, J, F), jnp.float32),
        grid=(n_tiles,),
        in_specs=[pl.BlockSpec((bt, 1, L), lambda t: (t, 0, 0)),
                  pl.BlockSpec((bt, J, D), lambda t: (t, 0, 0))]
                 + [_full(w) for w in weights],
        out_specs=pl.BlockSpec((bt, J, F), lambda t: (t, 0, 0)),
        compiler_params=pltpu.CompilerParams(dimension_semantics=("parallel",)),
    )(label, skt, *weights)
    if B_pad != B:
        out = out[:B]
    return out


# speedup vs baseline: 1.1183x; 1.0236x over previous
import jax
import jax.numpy as jnp
from jax.experimental import pallas as pl
from jax.experimental.pallas import tpu as pltpu

LN_EPS = 1e-5
NEG_BIG = -1e30

_HEADS = 8
_DH = 32
_DEPTH = 4


def _erf(x):
    a1, a2, a3, a4, a5 = 0.254829592, -0.284496736, 1.421413741, -1.453152027, 1.061405429
    p = 0.3275911
    s = jnp.sign(x)
    ax = jnp.abs(x)
    t = 1.0 / (1.0 + p * ax)
    poly = ((((a5 * t + a4) * t + a3) * t + a2) * t + a1) * t
    return s * (1.0 - poly * jnp.exp(-ax * ax))


def _gelu(x):
    return 0.5 * x * (1.0 + _erf(x * 0.7071067811865475))


def _layernorm(x, g, b):
    mu = jnp.mean(x, axis=-1, keepdims=True)
    xc = x - mu
    var = jnp.mean(xc * xc, axis=-1, keepdims=True)
    return xc * jax.lax.rsqrt(var + LN_EPS) * g + b


def _corrector_kernel(label_ref, skt_ref, pos_ref, wlab_ref, blab_ref, *refs):
    lrefs = refs[:11 * _DEPTH]
    gw01_ref, gm_ref, am0_ref, aoffr_ref, gb_ref, o_ref = refs[11 * _DEPTH:]

    bt = label_ref.shape[0]
    lab_dim = label_ref.shape[2]
    jt = skt_ref.shape[1]
    dm = skt_ref.shape[2]
    np_tok = pos_ref.shape[0]
    n_real = jt + 1
    fdim = gb_ref.shape[-1]
    hnp = _HEADS * np_tok
    scale = _DH ** -0.5

    lab = label_ref[...].reshape(bt, lab_dim)
    emb = jnp.dot(lab, wlab_ref[...], preferred_element_type=jnp.float32) + blab_ref[...]
    x3 = jnp.concatenate(
        [skt_ref[...], emb[:, None, :], jnp.zeros((bt, np_tok - n_real, dm), jnp.float32)],
        axis=1)
    x = (x3 + pos_ref[...][None]).reshape(bt * np_tok, dm)

    row_i = jax.lax.broadcasted_iota(jnp.int32, (hnp, 2 * dm), 0)
    col_i = jax.lax.broadcasted_iota(jnp.int32, (hnp, 2 * dm), 1)
    kvmask2 = ((row_i // np_tok) == ((col_i % dm) // _DH)).astype(jnp.float32)
    kvmask = kvmask2[:, :dm]
    cb = jax.lax.broadcasted_iota(jnp.int32, (1, 1, hnp), 2)
    colbias = jnp.where((cb % np_tok) < n_real, 0.0, NEG_BIG).astype(jnp.float32)

    for l in range(_DEPTH):
        (ln1g_ref, ln1b_ref, wqkv_ref, wo_ref, bo_ref,
         ln2g_ref, ln2b_ref, w1_ref, b1_ref, w2_ref, b2_ref) = lrefs[11 * l:11 * (l + 1)]
        xn = _layernorm(x, ln1g_ref[...], ln1b_ref[...])
        qkv = jnp.dot(xn, wqkv_ref[...], preferred_element_type=jnp.float32)
        qkv3 = qkv.reshape(bt, np_tok, 3 * dm)
        q = qkv3[..., :dm]
        kv = qkv3[..., dm:]
        kvhat = jnp.concatenate([kv] * _HEADS, axis=1) * kvmask2[None]
        khat = kvhat[..., :dm]
        vhat = kvhat[..., dm:]
        s = jax.lax.dot_general(q, khat, (((2,), (2,)), ((0,), (0,))),
                                preferred_element_type=jnp.float32)
        s = s * scale + colbias
        s = s - jnp.max(s, axis=2, keepdims=True)
        p = jnp.exp(s)
        pv = jax.lax.dot_general(p, vhat, (((2,), (1,)), ((0,), (0,))),
                                 preferred_element_type=jnp.float32)
        den = jnp.dot(p.reshape(bt * np_tok, hnp), kvmask,
                      preferred_element_type=jnp.float32)
        attn = pv.reshape(bt * np_tok, dm) / den
        x = x + jnp.dot(attn, wo_ref[...], preferred_element_type=jnp.float32) + bo_ref[...]

        xn2 = _layernorm(x, ln2g_ref[...], ln2b_ref[...])
        hid = _gelu(jnp.dot(xn2, w1_ref[...], preferred_element_type=jnp.float32) + b1_ref[...])
        x = x + jnp.dot(hid, w2_ref[...], preferred_element_type=jnp.float32) + b2_ref[...]

    xj = x.reshape(bt, np_tok, dm)[:, :jt, :].reshape(bt * jt, dm)
    h01 = jnp.dot(xj, gw01_ref[...], preferred_element_type=jnp.float32)
    h3 = h01.reshape(bt, jt, 2 * fdim)
    h0 = h3[..., :fdim]
    h1 = h3[..., fdim:]
    mh1 = gm_ref[...][None] * h1
    z = jnp.zeros((bt, jt, fdim), jnp.float32)
    for kk in range(jt):
        z = z + mh1[:, kk:kk + 1, :] * aoffr_ref[kk][None]
    o_ref[...] = am0_ref[...][None] * h0 + z + gb_ref[...][None]


def _full(arr):
    nd = arr.ndim
    return pl.BlockSpec(arr.shape, lambda t, _nd=nd: (0,) * _nd)


def kernel(label, skt, label_emb_w, label_emb_b, pos_embedding, gcn_w0, gcn_w1,
           gcn_m, adj, adj2, gcn_bias, *layer_args):
    B, _, L = label.shape
    J, D = skt.shape[1], skt.shape[2]
    N = J + 1
    Np = -(-N // 8) * 8
    F = gcn_bias.shape[-1]

    bt = min(64, B)
    n_tiles = -(-B // bt)
    B_pad = n_tiles * bt

    label = label.astype(jnp.float32)
    skt = skt.astype(jnp.float32)
    if B_pad != B:
        label = jnp.pad(label, ((0, B_pad - B), (0, 0), (0, 0)))
        skt = jnp.pad(skt, ((0, B_pad - B), (0, 0), (0, 0)))

    pos = pos_embedding[0]
    pos_perm = jnp.concatenate([pos[1:], pos[:1]], axis=0)
    pos_pad = jnp.zeros((Np, D), jnp.float32).at[:N].set(pos_perm)

    a = adj + adj2
    a_sym = 0.5 * (a + a.T)
    eye = jnp.eye(J, dtype=jnp.float32)
    a_off = a_sym * (1.0 - eye)
    am0 = jnp.diagonal(a_sym)[:, None] * gcn_m
    gw01 = jnp.concatenate([gcn_w0, gcn_w1], axis=1)
    aoff_rep = jnp.broadcast_to(a_off.T[:, :, None], (J, J, F))

    weights = ([pos_pad, label_emb_w, label_emb_b] + list(layer_args)
               + [gw01, gcn_m, am0, aoff_rep, gcn_bias])

    out = pl.pallas_call(
        _corrector_kernel,
        out_shape=jax.ShapeDtypeStruct((B_pad, J, F), jnp.float32),
        grid=(n_tiles,),
        in_specs=[pl.BlockSpec((bt, 1, L), lambda t: (t, 0, 0)),
                  pl.BlockSpec((bt, J, D), lambda t: (t, 0, 0))]
                 + [_full(w) for w in weights],
        out_specs=pl.BlockSpec((bt, J, F), lambda t: (t, 0, 0)),
        compiler_params=pltpu.CompilerParams(dimension_semantics=("parallel",)),
    )(label, skt, *weights)
    if B_pad != B:
        out = out[:B]
    return out


# speedup vs baseline: 2.0284x; 1.8139x over previous
import jax
import jax.numpy as jnp
from jax.experimental import pallas as pl
from jax.experimental.pallas import tpu as pltpu

LN_EPS = 1e-5
NEG_BIG = -1e30

_HEADS = 8
_DH = 32
_DEPTH = 4


def _gelu(x):
    inner = 0.7978845608028654 * x * (1.0 + 0.044715 * x * x)
    return 0.5 * x * (1.0 + jnp.tanh(inner))


def _layernorm(x, g, b):
    mu = jnp.mean(x, axis=-1, keepdims=True)
    xc = x - mu
    var = jnp.mean(xc * xc, axis=-1, keepdims=True)
    return xc * jax.lax.rsqrt(var + LN_EPS) * g + b


def _corrector_kernel(label_ref, skt_ref, pos_ref, wlab_ref, blab_ref, *refs):
    lrefs = refs[:11 * _DEPTH]
    gw01_ref, gm_ref, am0_ref, aoffr_ref, gb_ref, o_ref = refs[11 * _DEPTH:]

    bt = label_ref.shape[0]
    lab_dim = label_ref.shape[2]
    jt = skt_ref.shape[1]
    dm = skt_ref.shape[2]
    np_tok = pos_ref.shape[0]
    n_real = jt + 1
    fdim = gb_ref.shape[-1]
    hnp = _HEADS * np_tok
    scale = _DH ** -0.5

    lab = label_ref[...].reshape(bt, lab_dim)
    emb = jnp.dot(lab, wlab_ref[...], preferred_element_type=jnp.float32) + blab_ref[...]
    x3 = jnp.concatenate(
        [skt_ref[...], emb[:, None, :], jnp.zeros((bt, np_tok - n_real, dm), jnp.float32)],
        axis=1)
    x = (x3 + pos_ref[...][None]).reshape(bt * np_tok, dm)

    row_i = jax.lax.broadcasted_iota(jnp.int32, (hnp, 2 * dm), 0)
    col_i = jax.lax.broadcasted_iota(jnp.int32, (hnp, 2 * dm), 1)
    kvmask2 = ((row_i // np_tok) == ((col_i % dm) // _DH)).astype(jnp.bfloat16)
    kvmask = kvmask2[:, :dm]
    cb = jax.lax.broadcasted_iota(jnp.int32, (1, 1, hnp), 2)
    colbias = jnp.where((cb % np_tok) < n_real, 0.0, NEG_BIG).astype(jnp.float32)

    for l in range(_DEPTH):
        (ln1g_ref, ln1b_ref, wqkv_ref, wo_ref, bo_ref,
         ln2g_ref, ln2b_ref, w1_ref, b1_ref, w2_ref, b2_ref) = lrefs[11 * l:11 * (l + 1)]
        xn = _layernorm(x, ln1g_ref[...], ln1b_ref[...])
        qkv = jnp.dot(xn, wqkv_ref[...], preferred_element_type=jnp.float32)
        qkv3 = qkv.reshape(bt, np_tok, 3 * dm)
        q = qkv3[..., :dm]
        kv = qkv3[..., dm:].astype(jnp.bfloat16)
        kvhat = jnp.concatenate([kv] * _HEADS, axis=1) * kvmask2[None]
        khat = kvhat[..., :dm]
        vhat = kvhat[..., dm:]
        s = jax.lax.dot_general(q.astype(jnp.bfloat16), khat,
                                (((2,), (2,)), ((0,), (0,))),
                                preferred_element_type=jnp.float32)
        s = s * scale + colbias
        s = s - jnp.max(s, axis=2, keepdims=True)
        p = jnp.exp(s).astype(jnp.bfloat16)
        pv = jax.lax.dot_general(p, vhat, (((2,), (1,)), ((0,), (0,))),
                                 preferred_element_type=jnp.float32)
        den = jnp.dot(p.reshape(bt * np_tok, hnp), kvmask,
                      preferred_element_type=jnp.float32)
        attn = pv.reshape(bt * np_tok, dm) / den
        x = x + jnp.dot(attn, wo_ref[...], preferred_element_type=jnp.float32) + bo_ref[...]

        xn2 = _layernorm(x, ln2g_ref[...], ln2b_ref[...])
        hid = _gelu(jnp.dot(xn2, w1_ref[...], preferred_element_type=jnp.float32) + b1_ref[...])
        x = x + jnp.dot(hid, w2_ref[...], preferred_element_type=jnp.float32) + b2_ref[...]

    xj = x.reshape(bt, np_tok, dm)[:, :jt, :].reshape(bt * jt, dm)
    h01 = jnp.dot(xj, gw01_ref[...], preferred_element_type=jnp.float32)
    h3 = h01.reshape(bt, jt, 2 * fdim)
    h0 = h3[..., :fdim]
    h1 = h3[..., fdim:]
    mh1 = gm_ref[...][None] * h1
    z = jnp.zeros((bt, jt, fdim), jnp.float32)
    for kk in range(jt):
        z = z + mh1[:, kk:kk + 1, :] * aoffr_ref[kk][None]
    o_ref[...] = am0_ref[...][None] * h0 + z + gb_ref[...][None]


def _full(arr):
    nd = arr.ndim
    return pl.BlockSpec(arr.shape, lambda t, _nd=nd: (0,) * _nd)


def kernel(label, skt, label_emb_w, label_emb_b, pos_embedding, gcn_w0, gcn_w1,
           gcn_m, adj, adj2, gcn_bias, *layer_args):
    B, _, L = label.shape
    J, D = skt.shape[1], skt.shape[2]
    N = J + 1
    Np = -(-N // 8) * 8
    F = gcn_bias.shape[-1]

    bt = min(64, B)
    n_tiles = -(-B // bt)
    B_pad = n_tiles * bt

    label = label.astype(jnp.float32)
    skt = skt.astype(jnp.float32)
    if B_pad != B:
        label = jnp.pad(label, ((0, B_pad - B), (0, 0), (0, 0)))
        skt = jnp.pad(skt, ((0, B_pad - B), (0, 0), (0, 0)))

    pos = pos_embedding[0]
    pos_perm = jnp.concatenate([pos[1:], pos[:1]], axis=0)
    pos_pad = jnp.zeros((Np, D), jnp.float32).at[:N].set(pos_perm)

    a = adj + adj2
    a_sym = 0.5 * (a + a.T)
    eye = jnp.eye(J, dtype=jnp.float32)
    a_off = a_sym * (1.0 - eye)
    am0 = jnp.diagonal(a_sym)[:, None] * gcn_m
    gw01 = jnp.concatenate([gcn_w0, gcn_w1], axis=1)
    aoff_rep = jnp.broadcast_to(a_off.T[:, :, None], (J, J, F))

    weights = ([pos_pad, label_emb_w, label_emb_b] + list(layer_args)
               + [gw01, gcn_m, am0, aoff_rep, gcn_bias])

    out = pl.pallas_call(
        _corrector_kernel,
        out_shape=jax.ShapeDtypeStruct((B_pad, J, F), jnp.float32),
        grid=(n_tiles,),
        in_specs=[pl.BlockSpec((bt, 1, L), lambda t: (t, 0, 0)),
                  pl.BlockSpec((bt, J, D), lambda t: (t, 0, 0))]
                 + [_full(w) for w in weights],
        out_specs=pl.BlockSpec((bt, J, F), lambda t: (t, 0, 0)),
        compiler_params=pltpu.CompilerParams(dimension_semantics=("parallel",)),
    )(label, skt, *weights)
    if B_pad != B:
        out = out[:B]
    return out
